# Initial kernel scaffold; baseline (speedup 1.0000x reference)
import jax
import jax.numpy as jnp
from jax.experimental import pallas as pl


def kernel(input1, input2, eps, iters):
    raise NotImplementedError("write your pallas kernel here")


# speedup vs baseline: 2.9100x; 2.9100x over previous
import jax
import jax.numpy as jnp
from jax.experimental import pallas as pl
from jax.experimental.pallas import tpu as pltpu

_N = 1024


def _auction_body(eps_ref, iters_ref, x1_ref, x2t_ref, dist_ref, ass_ref, c_ref):
    n = _N
    x1 = x1_ref[0]
    x2t = x2t_ref[0]
    eps = eps_ref[0]
    iters = iters_ref[0]

    d0 = x1[:, 0:1] - x2t[0:1, :]
    d1 = x1[:, 1:2] - x2t[1:2, :]
    d2 = x1[:, 2:3] - x2t[2:3, :]
    c_ref[...] = (d0 * d0 + d1 * d1) + d2 * d2

    col = jax.lax.broadcasted_iota(jnp.int32, (1, n), 1)
    row = jax.lax.broadcasted_iota(jnp.int32, (n, 1), 0)
    neg_inf = jnp.float32(-jnp.inf)

    def body(_, carry):
        price, ass, ass_inv = carry
        c = c_ref[...]
        v = -c - price
        best = jnp.max(v, axis=1, keepdims=True)
        iseq = v == best
        cnt = jnp.sum(iseq.astype(jnp.int32), axis=1, keepdims=True)
        bidx = jnp.min(jnp.where(iseq, col, n), axis=1, keepdims=True)
        m2 = jnp.max(jnp.where(iseq, neg_inf, v), axis=1, keepdims=True)
        second = jnp.where(cnt > 1, best, m2)
        bid_inc = best - second + eps
        unass = ass < 0
        bmask = (bidx == col) & unass
        bids = jnp.where(bmask, bid_inc, neg_inf)
        max_inc = jnp.max(bids, axis=0, keepdims=True)
        winner = jnp.min(
            jnp.where(bids == max_inc, row, n), axis=0, keepdims=True
        )
        has_bid = max_inc > neg_inf
        clearm = has_bid & (ass_inv == row)
        clear = jnp.any(clearm, axis=1, keepdims=True)
        ass1 = jnp.where(clear, jnp.int32(-1), ass)
        am = has_bid & (winner == row)
        got = jnp.max(jnp.where(am, col, jnp.int32(-1)), axis=1, keepdims=True)
        ass2 = jnp.where(got >= 0, got, ass1)
        ass_inv2 = jnp.where(has_bid, winner, ass_inv)
        price2 = jnp.where(has_bid, price + max_inc, price)
        return price2, ass2, ass_inv2

    price0 = jnp.zeros((1, n), jnp.float32)
    ass0 = jnp.full((n, 1), -1, jnp.int32)
    ass_inv0 = jnp.full((1, n), -1, jnp.int32)
    _, ass, _ = jax.lax.fori_loop(0, iters, body, (price0, ass0, ass_inv0))

    dm = ass == col
    dist = jnp.max(jnp.where(dm, c_ref[...], 0.0), axis=1, keepdims=True)
    dist_ref[0] = dist
    ass_ref[0] = ass


def kernel(input1, input2, eps, iters):
    b, n, _ = input1.shape
    x2t = jnp.transpose(input2, (0, 2, 1))
    eps_a = jnp.asarray(eps, jnp.float32).reshape(1)
    it_a = jnp.asarray(iters, jnp.int32).reshape(1)
    dist3, ass3 = pl.pallas_call(
        _auction_body,
        grid=(b,),
        in_specs=[
            pl.BlockSpec(memory_space=pltpu.SMEM),
            pl.BlockSpec(memory_space=pltpu.SMEM),
            pl.BlockSpec((1, n, 3), lambda i: (i, 0, 0)),
            pl.BlockSpec((1, 3, n), lambda i: (i, 0, 0)),
        ],
        out_specs=[
            pl.BlockSpec((1, n, 1), lambda i: (i, 0, 0)),
            pl.BlockSpec((1, n, 1), lambda i: (i, 0, 0)),
        ],
        out_shape=[
            jax.ShapeDtypeStruct((b, n, 1), jnp.float32),
            jax.ShapeDtypeStruct((b, n, 1), jnp.int32),
        ],
        scratch_shapes=[pltpu.VMEM((n, n), jnp.float32)],
    )(eps_a, it_a, input1, x2t)
    return dist3[..., 0], ass3[..., 0]
